# baseline (speedup 1.0000x reference)
import functools

import jax
import jax.numpy as jnp
from jax import lax
from jax.experimental import pallas as pl
from jax.experimental.pallas import tpu as pltpu
from jax.experimental.pallas import tpu_sc as plsc

VOCAB = 1000000
DIM = 64
B = 16384

NUM_CORES = 2
NUM_SUBCORES = 16
LANES = 16
NW = NUM_CORES * NUM_SUBCORES
BPW = B // NW
CHUNK = 128
NCHUNK = BPW // CHUNK


def _sc_body(tgt_hbm, ctx_hbm, table_hbm, out_hbm,
             idx_t, idx_c, u_rows, v_rows, out_v, tmp, sem):
    wid = lax.axis_index("s") * NUM_CORES + lax.axis_index("c")
    base = wid * BPW

    pltpu.sync_copy(tgt_hbm.at[pl.ds(base, BPW)], idx_t)
    pltpu.sync_copy(ctx_hbm.at[pl.ds(base, BPW)], idx_c)

    copies = []
    for j in range(NCHUNK):
        sl = pl.ds(j * CHUNK, CHUNK)
        copies.append(pltpu.async_copy(table_hbm.at[idx_t.at[sl]], u_rows.at[sl], sem))
        copies.append(pltpu.async_copy(table_hbm.at[idx_c.at[sl]], v_rows.at[sl], sem))
    for cp in copies:
        cp.wait()

    rowbase = lax.iota(jnp.int32, 16) * LANES

    def group(g, carry):
        for rr in range(LANES):
            r = g * LANES + rr
            acc = jnp.zeros((LANES,), jnp.float32)
            for c in range(DIM // LANES):
                u = u_rows[r, pl.ds(c * LANES, LANES)]
                v = v_rows[r, pl.ds(c * LANES, LANES)]
                acc = acc + u * v
            tmp[pl.ds(rr * LANES, LANES)] = acc
        res = jnp.zeros((LANES,), jnp.float32)
        for jj in range(LANES):
            res = res + plsc.load_gather(tmp, [rowbase + jj])
        out_v[pl.ds(g * LANES, LANES)] = res
        return carry

    lax.fori_loop(0, BPW // LANES, group, 0)

    pltpu.sync_copy(out_v, out_hbm.at[pl.ds(base, BPW)])


@jax.jit
def _skipgram(target, context, table):
    mesh = plsc.VectorSubcoreMesh(core_axis_name="c", subcore_axis_name="s")
    return pl.kernel(
        _sc_body,
        out_type=jax.ShapeDtypeStruct((B,), jnp.float32),
        mesh=mesh,
        scratch_types=[
            pltpu.VMEM((BPW,), jnp.int32),
            pltpu.VMEM((BPW,), jnp.int32),
            pltpu.VMEM((BPW, DIM), jnp.float32),
            pltpu.VMEM((BPW, DIM), jnp.float32),
            pltpu.VMEM((BPW,), jnp.float32),
            pltpu.VMEM((LANES * LANES,), jnp.float32),
            pltpu.SemaphoreType.DMA,
        ],
        compiler_params=pltpu.CompilerParams(
            needs_layout_passes=False, use_tc_tiling_on_sc=False),
    )(target, context, table)


def kernel(target, context, embedding_weights):
    return _skipgram(target.astype(jnp.int32), context.astype(jnp.int32),
                     embedding_weights)
